# baseline (speedup 1.0000x reference)
import functools

import jax
import jax.numpy as jnp
from jax import lax
from jax.experimental import pallas as pl
from jax.experimental.pallas import tpu as pltpu
from jax.experimental.pallas import tpu_sc as plsc

B = 4096
Q_LEN = 20
D_LEN = 200
EDIM = 32

NC = 2
NS = 16
NW = NC * NS
GB = B // NW
QT = EDIM // 8
DT = D_LEN // 8
NR = 4

_mesh = plsc.VectorSubcoreMesh(core_axis_name="c", subcore_axis_name="s")


@functools.partial(
    pl.kernel,
    mesh=_mesh,
    out_type=[
        jax.ShapeDtypeStruct((Q_LEN, QT, NW, 8, GB), jnp.float32),
        jax.ShapeDtypeStruct((EDIM, DT, NW, 8, GB), jnp.float32),
    ],
    scratch_types=[
        pltpu.VMEM((GB, Q_LEN), jnp.int32),
        pltpu.VMEM((2, GB, 8), jnp.int32),
        pltpu.VMEM((Q_LEN, GB), jnp.int32),
        pltpu.VMEM((D_LEN, GB), jnp.int32),
        pltpu.VMEM((NR, GB, EDIM), jnp.float32),
        pltpu.VMEM((2, QT, 8, GB), jnp.float32),
        pltpu.VMEM((2, EDIM, 8, GB), jnp.float32),
        [pltpu.SemaphoreType.DMA] * NR,
        [pltpu.SemaphoreType.DMA] * 2,
        [pltpu.SemaphoreType.DMA] * 2,
        [pltpu.SemaphoreType.DMA] * 2,
    ],
    compiler_params=pltpu.CompilerParams(
        needs_layout_passes=False, use_tc_tiling_on_sc=False),
)
def _emb_kernel(q_hbm, d_hbm, table_hbm, q_out_hbm, d_out_hbm,
                qraw_v, dchunk_v, qidx_v, didx_v, rows_v, qstage_v, dstage_v,
                gsems, qosems, dosems, csems):
    wid = lax.axis_index("s") * NC + lax.axis_index("c")
    b0 = wid * GB
    ri = lax.iota(jnp.int32, 16)

    pltpu.sync_copy(q_hbm.at[pl.ds(b0, GB)], qraw_v)

    def issue_chunk(t, s):
        pltpu.async_copy(
            d_hbm.at[pl.ds(b0, GB), pl.ds(t * 8, 8)], dchunk_v.at[s],
            csems[s])

    def wait_chunk(s):
        pltpu.make_async_copy(
            d_hbm.at[pl.ds(0, GB), pl.ds(0, 8)], dchunk_v.at[s],
            csems[s]).wait()

    issue_chunk(0, 0)
    issue_chunk(1, 1)

    def tq_body(l, carry):
        lcol = jnp.full((16,), l, jnp.int32)

        def qc_body(c, carry2):
            v = plsc.load_gather(qraw_v, [c * 16 + ri, lcol])
            qidx_v[l, pl.ds(c * 16, 16)] = v
            return carry2

        lax.fori_loop(0, GB // 16, qc_body, 0)
        return carry

    lax.fori_loop(0, Q_LEN, tq_body, 0)

    def td_chunk(t, s):
        wait_chunk(s)

        def dl_body(li, carry):
            lcol = jnp.full((16,), li, jnp.int32)

            def dc_body(c, carry2):
                v = plsc.load_gather(dchunk_v.at[s], [c * 16 + ri, lcol])
                didx_v[t * 8 + li, pl.ds(c * 16, 16)] = v
                return carry2

            lax.fori_loop(0, GB // 16, dc_body, 0)
            return carry

        lax.fori_loop(0, 8, dl_body, 0)

    def td_body(j, carry):
        for s in range(2):
            t = 2 * j + s
            td_chunk(t, s)
            @pl.when(t + 2 < DT)
            def _():
                issue_chunk(t + 2, s)
        return carry

    lax.fori_loop(0, DT // 2, td_body, 0)
    td_chunk(DT - 1, 0)

    def issue_q(l, r):
        pltpu.async_copy(table_hbm.at[qidx_v.at[l]], rows_v.at[r], gsems[r])

    def issue_d(l, r):
        pltpu.async_copy(table_hbm.at[didx_v.at[l]], rows_v.at[r], gsems[r])

    def wait_g(r):
        pltpu.make_async_copy(
            table_hbm.at[qidx_v.at[0]], rows_v.at[r], gsems[r]).wait()

    def transpose_into(r, store):
        def cbody(c, carry):
            bi = c * 16 + ri
            for d in range(EDIM):
                col = jnp.full((16,), d, jnp.int32)
                v = plsc.load_gather(rows_v.at[r], [bi, col])
                store(d, c * 16, v)
            return carry
        lax.fori_loop(0, GB // 16, cbody, 0)

    issue_q(0, 0)
    issue_q(1, 1)

    def qbody(j, carry):
        for s in range(2):
            l = 2 * j + s
            wait_g(s)

            @pl.when(j > 0)
            def _():
                pltpu.make_async_copy(
                    qstage_v.at[s], q_out_hbm.at[0, :, 0], qosems[s]).wait()

            qstage = qstage_v.at[s]
            transpose_into(
                s, lambda d, c0, v: qstage.__setitem__(
                    (d // 8, d % 8, pl.ds(c0, 16)), v))
            issue_q(jnp.minimum(l + 2, Q_LEN - 1), s)
            pltpu.async_copy(
                qstage_v.at[s], q_out_hbm.at[l, :, wid], qosems[s])
        return carry

    lax.fori_loop(0, Q_LEN // 2, qbody, 0)
    for s in range(2):
        wait_g(s)
        pltpu.make_async_copy(
            qstage_v.at[s], q_out_hbm.at[0, :, 0], qosems[s]).wait()

    for r in range(NR):
        issue_d(r, r)

    def wait_do(sd):
        pltpu.make_async_copy(
            dstage_v.at[sd], d_out_hbm.at[:, 0, 0], dosems[sd]).wait()

    def do_tile(lt, sd):
        dstage = dstage_v.at[sd]
        for li in range(8):
            r = li % NR
            l = lt * 8 + li
            wait_g(r)
            transpose_into(
                r, lambda d, c0, v: dstage.__setitem__(
                    (d, li, pl.ds(c0, 16)), v))
            issue_d(jnp.minimum(l + NR, D_LEN - 1), r)
        pltpu.async_copy(
            dstage_v.at[sd], d_out_hbm.at[:, lt, wid], dosems[sd])

    def dbody(j, carry):
        for sd in range(2):
            @pl.when(j > 0)
            def _():
                wait_do(sd)
            do_tile(2 * j + sd, sd)
        return carry

    lax.fori_loop(0, DT // 2, dbody, 0)
    wait_do(0)
    do_tile(DT - 1, 0)

    wait_do(0)
    wait_do(1)
    for r in range(NR):
        wait_g(r)


def kernel(query_input, document_input, table):
    QO, DO = _emb_kernel(query_input, document_input, table)
    q_out = jnp.transpose(QO, (2, 4, 1, 3, 0)).reshape(B, EDIM, Q_LEN)
    d_out = jnp.transpose(DO, (2, 4, 0, 1, 3)).reshape(B, EDIM, D_LEN)
    return (q_out, d_out)


# speedup vs baseline: 1.5004x; 1.5004x over previous
import functools

import jax
import jax.numpy as jnp
from jax import lax
from jax.experimental import pallas as pl
from jax.experimental.pallas import tpu as pltpu
from jax.experimental.pallas import tpu_sc as plsc

B = 4096
Q_LEN = 20
D_LEN = 200
EDIM = 32

NC = 2
NS = 16
NW = NC * NS
GB = B // NW
QT = EDIM // 8
DT = D_LEN // 8
NR = 4

_mesh = plsc.VectorSubcoreMesh(core_axis_name="c", subcore_axis_name="s")


@functools.partial(
    pl.kernel,
    mesh=_mesh,
    out_type=[
        jax.ShapeDtypeStruct((Q_LEN, QT, NW, 8, GB), jnp.float32),
        jax.ShapeDtypeStruct((EDIM, DT, NW, 8, GB), jnp.float32),
    ],
    scratch_types=[
        pltpu.VMEM((GB, Q_LEN), jnp.int32),
        pltpu.VMEM((2, GB, 8), jnp.int32),
        pltpu.VMEM((Q_LEN, GB), jnp.int32),
        pltpu.VMEM((D_LEN, GB), jnp.int32),
        pltpu.VMEM((NR, GB, EDIM), jnp.float32),
        pltpu.VMEM((2, QT, 8, GB), jnp.float32),
        pltpu.VMEM((2, EDIM, 8, GB), jnp.float32),
        [pltpu.SemaphoreType.DMA] * NR,
        [pltpu.SemaphoreType.DMA] * 2,
        [pltpu.SemaphoreType.DMA] * 2,
        [pltpu.SemaphoreType.DMA] * 2,
    ],
    compiler_params=pltpu.CompilerParams(
        needs_layout_passes=False, use_tc_tiling_on_sc=False),
)
def _emb_kernel(q_hbm, d_hbm, table_hbm, q_out_hbm, d_out_hbm,
                qraw_v, dchunk_v, qidx_v, didx_v, rows_v, qstage_v, dstage_v,
                gsems, qosems, dosems, csems):
    wid = lax.axis_index("s") * NC + lax.axis_index("c")
    b0 = wid * GB
    ri = lax.iota(jnp.int32, 16)

    pltpu.sync_copy(q_hbm.at[pl.ds(b0, GB)], qraw_v)

    def issue_chunk(t, s):
        pltpu.async_copy(
            d_hbm.at[pl.ds(b0, GB), pl.ds(t * 8, 8)], dchunk_v.at[s],
            csems[s])

    def wait_chunk(s):
        pltpu.make_async_copy(
            d_hbm.at[pl.ds(0, GB), pl.ds(0, 8)], dchunk_v.at[s],
            csems[s]).wait()

    issue_chunk(0, 0)
    issue_chunk(1, 1)

    def tq_body(l, carry):
        lcol = jnp.full((16,), l, jnp.int32)

        def qc_body(c, carry2):
            v = plsc.load_gather(qraw_v, [c * 16 + ri, lcol])
            qidx_v[l, pl.ds(c * 16, 16)] = v
            return carry2

        lax.fori_loop(0, GB // 16, qc_body, 0)
        return carry

    lax.fori_loop(0, Q_LEN, tq_body, 0)

    def td_chunk(t, s):
        wait_chunk(s)

        def dl_body(li, carry):
            lcol = jnp.full((16,), li, jnp.int32)

            def dc_body(c, carry2):
                v = plsc.load_gather(dchunk_v.at[s], [c * 16 + ri, lcol])
                didx_v[t * 8 + li, pl.ds(c * 16, 16)] = v
                return carry2

            lax.fori_loop(0, GB // 16, dc_body, 0)
            return carry

        lax.fori_loop(0, 8, dl_body, 0)

    def td_body(j, carry):
        for s in range(2):
            t = 2 * j + s
            td_chunk(t, s)
            @pl.when(t + 2 < DT)
            def _():
                issue_chunk(t + 2, s)
        return carry

    lax.fori_loop(0, DT // 2, td_body, 0)
    td_chunk(DT - 1, 0)

    def issue_q(l, r):
        pltpu.async_copy(table_hbm.at[qidx_v.at[l]], rows_v.at[r], gsems[r])

    def issue_d(l, r):
        pltpu.async_copy(table_hbm.at[didx_v.at[l]], rows_v.at[r], gsems[r])

    def wait_g(r):
        pltpu.make_async_copy(
            table_hbm.at[qidx_v.at[0]], rows_v.at[r], gsems[r]).wait()

    def transpose_into(r, store_vec):
        def kbody(k, carry):
            dvec = (k + ri) & (EDIM - 1)

            def cbody(c, carry2):
                bidx = c * 16 + ri
                v = plsc.load_gather(rows_v.at[r], [bidx, dvec])
                store_vec(dvec, bidx, v)
                return carry2

            lax.fori_loop(0, GB // 16, cbody, 0)
            return carry

        lax.fori_loop(0, EDIM, kbody, 0)

    issue_q(0, 0)
    issue_q(1, 1)

    def qbody(j, carry):
        for s in range(2):
            l = 2 * j + s
            wait_g(s)

            @pl.when(j > 0)
            def _():
                pltpu.make_async_copy(
                    qstage_v.at[s], q_out_hbm.at[0, :, 0], qosems[s]).wait()

            qstage = qstage_v.at[s]
            transpose_into(
                s, lambda dvec, bidx, v: plsc.store_scatter(
                    qstage, [dvec // 8, dvec % 8, bidx], v))
            issue_q(jnp.minimum(l + 2, Q_LEN - 1), s)
            pltpu.async_copy(
                qstage_v.at[s], q_out_hbm.at[l, :, wid], qosems[s])
        return carry

    lax.fori_loop(0, Q_LEN // 2, qbody, 0)
    for s in range(2):
        wait_g(s)
        pltpu.make_async_copy(
            qstage_v.at[s], q_out_hbm.at[0, :, 0], qosems[s]).wait()

    for r in range(NR):
        issue_d(r, r)

    def wait_do(sd):
        pltpu.make_async_copy(
            dstage_v.at[sd], d_out_hbm.at[:, 0, 0], dosems[sd]).wait()

    def do_tile(lt, sd):
        dstage = dstage_v.at[sd]
        for li in range(8):
            r = li % NR
            l = lt * 8 + li
            livec = jnp.full((16,), li, jnp.int32)
            wait_g(r)
            transpose_into(
                r, lambda dvec, bidx, v: plsc.store_scatter(
                    dstage, [dvec, livec, bidx], v))
            issue_d(jnp.minimum(l + NR, D_LEN - 1), r)
        pltpu.async_copy(
            dstage_v.at[sd], d_out_hbm.at[:, lt, wid], dosems[sd])

    def dbody(j, carry):
        for sd in range(2):
            @pl.when(j > 0)
            def _():
                wait_do(sd)
            do_tile(2 * j + sd, sd)
        return carry

    lax.fori_loop(0, DT // 2, dbody, 0)
    wait_do(0)
    do_tile(DT - 1, 0)

    wait_do(0)
    wait_do(1)
    for r in range(NR):
        wait_g(r)


def kernel(query_input, document_input, table):
    QO, DO = _emb_kernel(query_input, document_input, table)
    q_out = jnp.transpose(QO, (2, 4, 1, 3, 0)).reshape(B, EDIM, Q_LEN)
    d_out = jnp.transpose(DO, (2, 4, 0, 1, 3)).reshape(B, EDIM, D_LEN)
    return (q_out, d_out)
